# baseline (speedup 1.0000x reference)
import functools

import jax
import jax.numpy as jnp
from jax import lax
from jax.experimental import pallas as pl
from jax.experimental.pallas import tpu as pltpu
from jax.experimental.pallas import tpu_sc as plsc

D_MODEL = 64
D_PAD = 128
SCALE = 8.0

_info = plsc.get_sparse_core_info()
_NC, _NS, _L = _info.num_cores, _info.num_subcores, _info.num_lanes
_NW = _NC * _NS

CHUNK = 128


def _gather_body(idx_hbm, table_hbm, out_hbm, idx_v, buf0, buf1, sem0, sem1,
                 *, b_per_w, n_chunks):
    wid = lax.axis_index("s") * _NC + lax.axis_index("c")
    base = wid * b_per_w
    pltpu.sync_copy(idx_hbm.at[pl.ds(base, b_per_w)], idx_v)

    bufs = (buf0, buf1)
    sems = (sem0, sem1)

    def start_gather(g, b):
        pltpu.make_async_copy(
            table_hbm.at[idx_v.at[pl.ds(g * CHUNK, CHUNK)]], bufs[b], sems[b]
        ).start()

    def finish(g, b):
        pltpu.make_async_copy(
            table_hbm.at[idx_v.at[pl.ds(g * CHUNK, CHUNK)]], bufs[b], sems[b]
        ).wait()
        pltpu.sync_copy(bufs[b], out_hbm.at[pl.ds(base + g * CHUNK, CHUNK)])

    start_gather(0, 0)
    start_gather(1, 1)

    def body(p, carry):
        g = p * 2
        finish(g, 0)

        @pl.when(g + 2 < n_chunks)
        def _():
            start_gather(g + 2, 0)

        finish(g + 1, 1)

        @pl.when(g + 3 < n_chunks)
        def _():
            start_gather(g + 3, 1)

        return carry

    lax.fori_loop(0, n_chunks // 2, body, 0)


def kernel(tokens, table):
    idx = tokens.reshape(-1).astype(jnp.int32)
    b_total = idx.shape[0]
    b_per_w = b_total // _NW
    n_chunks = b_per_w // CHUNK
    table8 = jnp.pad(table * jnp.float32(SCALE), ((0, 0), (0, D_PAD - D_MODEL)))
    mesh = plsc.VectorSubcoreMesh(core_axis_name="c", subcore_axis_name="s")
    out = pl.kernel(
        functools.partial(_gather_body, b_per_w=b_per_w, n_chunks=n_chunks),
        out_type=jax.ShapeDtypeStruct((b_total, D_PAD), jnp.float32),
        mesh=mesh,
        scratch_types=[
            pltpu.VMEM((b_per_w,), jnp.int32),
            pltpu.VMEM((CHUNK, D_PAD), jnp.float32),
            pltpu.VMEM((CHUNK, D_PAD), jnp.float32),
            pltpu.SemaphoreType.DMA,
            pltpu.SemaphoreType.DMA,
        ],
        compiler_params=pltpu.CompilerParams(use_tc_tiling_on_sc=True),
    )(idx, table8)
    return out[:, :D_MODEL].reshape(tokens.shape + (D_MODEL,))


# speedup vs baseline: 1.3180x; 1.3180x over previous
import functools

import jax
import jax.numpy as jnp
from jax import lax
from jax.experimental import pallas as pl
from jax.experimental.pallas import tpu as pltpu
from jax.experimental.pallas import tpu_sc as plsc

D_MODEL = 64
D_PAD = 128
SCALE = 8.0

_info = plsc.get_sparse_core_info()
_NC, _NS, _L = _info.num_cores, _info.num_subcores, _info.num_lanes
_NW = _NC * _NS

CHUNK = 128
BV = 2048


def _prep_body(tt_ref, out_ref):
    at = jnp.transpose(tt_ref[...]) * SCALE
    out_ref[...] = jnp.concatenate([at, at], axis=1)


def _prep_table(table):
    v = table.shape[0]
    tt = table.T
    return pl.pallas_call(
        _prep_body,
        grid=(pl.cdiv(v, BV),),
        in_specs=[pl.BlockSpec((D_MODEL, BV), lambda i: (0, i))],
        out_specs=pl.BlockSpec((BV, D_PAD), lambda i: (i, 0)),
        out_shape=jax.ShapeDtypeStruct((v, D_PAD), jnp.float32),
    )(tt)


def _gather_body(idx_hbm, table_hbm, out_hbm, idx_v, buf0, buf1, sem0, sem1,
                 *, b_per_w, n_chunks):
    wid = lax.axis_index("s") * _NC + lax.axis_index("c")
    base = wid * b_per_w
    pltpu.sync_copy(idx_hbm.at[pl.ds(base, b_per_w)], idx_v)

    bufs = (buf0, buf1)
    sems = (sem0, sem1)

    def start_gather(g, b):
        pltpu.make_async_copy(
            table_hbm.at[idx_v.at[pl.ds(g * CHUNK, CHUNK)]], bufs[b], sems[b]
        ).start()

    def finish(g, b):
        pltpu.make_async_copy(
            table_hbm.at[idx_v.at[pl.ds(g * CHUNK, CHUNK)]], bufs[b], sems[b]
        ).wait()
        pltpu.sync_copy(bufs[b], out_hbm.at[pl.ds(base + g * CHUNK, CHUNK)])

    start_gather(0, 0)
    start_gather(1, 1)

    def body(p, carry):
        g = p * 2
        finish(g, 0)

        @pl.when(g + 2 < n_chunks)
        def _():
            start_gather(g + 2, 0)

        finish(g + 1, 1)

        @pl.when(g + 3 < n_chunks)
        def _():
            start_gather(g + 3, 1)

        return carry

    lax.fori_loop(0, n_chunks // 2, body, 0)


def kernel(tokens, table):
    idx = tokens.reshape(-1).astype(jnp.int32)
    b_total = idx.shape[0]
    b_per_w = b_total // _NW
    n_chunks = b_per_w // CHUNK
    table8 = _prep_table(table)
    mesh = plsc.VectorSubcoreMesh(core_axis_name="c", subcore_axis_name="s")
    out = pl.kernel(
        functools.partial(_gather_body, b_per_w=b_per_w, n_chunks=n_chunks),
        out_type=jax.ShapeDtypeStruct((b_total, D_PAD), jnp.float32),
        mesh=mesh,
        scratch_types=[
            pltpu.VMEM((b_per_w,), jnp.int32),
            pltpu.VMEM((CHUNK, D_PAD), jnp.float32),
            pltpu.VMEM((CHUNK, D_PAD), jnp.float32),
            pltpu.SemaphoreType.DMA,
            pltpu.SemaphoreType.DMA,
        ],
        compiler_params=pltpu.CompilerParams(use_tc_tiling_on_sc=True),
    )(idx, table8)
    return out[:, :D_MODEL].reshape(tokens.shape + (D_MODEL,))
